# baseline (speedup 1.0000x reference)
import functools

import jax
import jax.numpy as jnp
from jax import lax
from jax.experimental import pallas as pl
from jax.experimental.pallas import tpu as pltpu
from jax.experimental.pallas import tpu_sc as plsc

INPUT_DIM = 4096
HIDDEN_DIMS = [8192, 8192]
OUTPUT_DIM = 4096
FANIN = 64
LAYER_DIMS = [INPUT_DIM] + HIDDEN_DIMS + [OUTPUT_DIM]
LAYER_INDICES = [0]
for _d in LAYER_DIMS:
    LAYER_INDICES.append(LAYER_INDICES[-1] + _d)

NC = 2
NS = 16
NW = NC * NS
LANES = 16


def _make_layer_kernel(n, prev_n, prev_start):
    npw = n // NW
    epw = npw * FANIN
    mesh = plsc.VectorSubcoreMesh(core_axis_name="c", subcore_axis_name="s")

    @functools.partial(
        pl.kernel,
        out_type=jax.ShapeDtypeStruct((n,), jnp.float32),
        mesh=mesh,
        compiler_params=pltpu.CompilerParams(needs_layout_passes=False),
        scratch_types=[
            pltpu.VMEM((prev_n,), jnp.float32),
            pltpu.VMEM((epw,), jnp.int32),
            pltpu.VMEM((epw,), jnp.float32),
            pltpu.VMEM((npw,), jnp.float32),
            pltpu.VMEM((npw,), jnp.float32),
        ],
    )
    def layer(prev_hbm, src_hbm, w_hbm, b_hbm, out_hbm,
              acts_v, src_v, w_v, b_v, out_v):
        wid = lax.axis_index("s") * NC + lax.axis_index("c")
        e_base = wid * epw
        n_base = wid * npw
        pltpu.sync_copy(prev_hbm, acts_v)
        pltpu.sync_copy(src_hbm.at[pl.ds(e_base, epw)], src_v)
        pltpu.sync_copy(w_hbm.at[pl.ds(e_base, epw)], w_v)
        pltpu.sync_copy(b_hbm.at[pl.ds(n_base, npw)], b_v)

        lane = lax.iota(jnp.int32, LANES)

        def group(g, _):
            pos0 = (g * LANES + lane) * FANIN
            accs = [jnp.zeros((LANES,), jnp.float32) for _ in range(4)]
            for j in range(FANIN):
                pos = pos0 + j
                s = plsc.load_gather(src_v, [pos])
                w = plsc.load_gather(w_v, [pos])
                a = plsc.load_gather(acts_v, [s - prev_start])
                accs[j % 4] = accs[j % 4] + a * w
            acc = (accs[0] + accs[1]) + (accs[2] + accs[3])
            b = b_v[pl.ds(g * LANES, LANES)]
            out_v[pl.ds(g * LANES, LANES)] = jnp.maximum(acc + b, 0.0)
            return 0

        lax.fori_loop(0, npw // LANES, group, 0)
        pltpu.sync_copy(out_v, out_hbm.at[pl.ds(n_base, npw)])

    return layer


_LAYER_KERNELS = []
for _i in range(1, len(LAYER_DIMS)):
    _LAYER_KERNELS.append(
        _make_layer_kernel(LAYER_DIMS[_i], LAYER_DIMS[_i - 1],
                           LAYER_INDICES[_i - 1]))


def kernel(x, edge_index, weights, bias):
    src = edge_index[0]
    acts = x
    offset = 0
    for i in range(1, len(LAYER_DIMS)):
        n = LAYER_DIMS[i]
        n_e = n * FANIN
        cur_start = LAYER_INDICES[i]
        acts = _LAYER_KERNELS[i - 1](
            acts,
            src[offset:offset + n_e],
            weights[offset:offset + n_e],
            bias[cur_start - INPUT_DIM:cur_start - INPUT_DIM + n],
        )
        offset += n_e
    return acts


# speedup vs baseline: 1.1769x; 1.1769x over previous
import functools

import jax
import jax.numpy as jnp
from jax import lax
from jax.experimental import pallas as pl
from jax.experimental.pallas import tpu as pltpu
from jax.experimental.pallas import tpu_sc as plsc

INPUT_DIM = 4096
HIDDEN_DIMS = [8192, 8192]
OUTPUT_DIM = 4096
FANIN = 64
LAYER_DIMS = [INPUT_DIM] + HIDDEN_DIMS + [OUTPUT_DIM]
LAYER_INDICES = [0]
for _d in LAYER_DIMS:
    LAYER_INDICES.append(LAYER_INDICES[-1] + _d)

NC = 2
NS = 16
NW = NC * NS
LANES = 16


def _make_layer_kernel(n, prev_n, prev_start):
    npw = n // NW
    epw = npw * FANIN
    mesh = plsc.VectorSubcoreMesh(core_axis_name="c", subcore_axis_name="s")

    @functools.partial(
        pl.kernel,
        out_type=jax.ShapeDtypeStruct((n,), jnp.float32),
        mesh=mesh,
        compiler_params=pltpu.CompilerParams(needs_layout_passes=False),
        scratch_types=[
            pltpu.VMEM((prev_n,), jnp.float32),
            pltpu.VMEM((epw,), jnp.int32),
            pltpu.VMEM((epw,), jnp.float32),
            pltpu.VMEM((npw,), jnp.float32),
            pltpu.VMEM((npw,), jnp.float32),
            pltpu.SemaphoreType.DMA,
            pltpu.SemaphoreType.DMA,
            pltpu.SemaphoreType.DMA,
            pltpu.SemaphoreType.DMA,
        ],
    )
    def layer(prev_hbm, src_hbm, w_hbm, b_hbm, out_hbm,
              acts_v, src_v, w_v, b_v, out_v, s0, s1, s2, s3):
        wid = lax.axis_index("s") * NC + lax.axis_index("c")
        e_base = wid * epw
        n_base = wid * npw
        c0 = pltpu.async_copy(prev_hbm, acts_v, s0)
        c1 = pltpu.async_copy(src_hbm.at[pl.ds(e_base, epw)], src_v, s1)
        c2 = pltpu.async_copy(w_hbm.at[pl.ds(e_base, epw)], w_v, s2)
        c3 = pltpu.async_copy(b_hbm.at[pl.ds(n_base, npw)], b_v, s3)
        c0.wait()
        c1.wait()
        c2.wait()
        c3.wait()

        lane = lax.iota(jnp.int32, LANES)
        zero = jnp.zeros((LANES,), jnp.float32)

        def group(g, _):
            pos0 = (g * LANES + lane) * FANIN

            def jblock(t, accs):
                base = pos0 + t * 8
                accs = list(accs)
                for u in range(8):
                    pos = base + u
                    s = plsc.load_gather(src_v, [pos])
                    w = plsc.load_gather(w_v, [pos])
                    a = plsc.load_gather(acts_v, [s - prev_start])
                    accs[u % 4] = accs[u % 4] + a * w
                return tuple(accs)

            accs = lax.fori_loop(0, FANIN // 8, jblock,
                                 (zero, zero, zero, zero))
            acc = (accs[0] + accs[1]) + (accs[2] + accs[3])
            b = b_v[pl.ds(g * LANES, LANES)]
            out_v[pl.ds(g * LANES, LANES)] = jnp.maximum(acc + b, 0.0)
            return 0

        lax.fori_loop(0, npw // LANES, group, 0)
        pltpu.sync_copy(out_v, out_hbm.at[pl.ds(n_base, npw)])

    return layer


_LAYER_KERNELS = []
for _i in range(1, len(LAYER_DIMS)):
    _LAYER_KERNELS.append(
        _make_layer_kernel(LAYER_DIMS[_i], LAYER_DIMS[_i - 1],
                           LAYER_INDICES[_i - 1]))


def kernel(x, edge_index, weights, bias):
    src = edge_index[0]
    acts = x
    offset = 0
    for i in range(1, len(LAYER_DIMS)):
        n = LAYER_DIMS[i]
        n_e = n * FANIN
        cur_start = LAYER_INDICES[i]
        acts = _LAYER_KERNELS[i - 1](
            acts,
            src[offset:offset + n_e],
            weights[offset:offset + n_e],
            bias[cur_start - INPUT_DIM:cur_start - INPUT_DIM + n],
        )
        offset += n_e
    return acts


# speedup vs baseline: 1.6644x; 1.4142x over previous
import functools

import jax
import jax.numpy as jnp
from jax import lax
from jax.experimental import pallas as pl
from jax.experimental.pallas import tpu as pltpu
from jax.experimental.pallas import tpu_sc as plsc

INPUT_DIM = 4096
HIDDEN_DIMS = [8192, 8192]
OUTPUT_DIM = 4096
FANIN = 64
LAYER_DIMS = [INPUT_DIM] + HIDDEN_DIMS + [OUTPUT_DIM]
LAYER_INDICES = [0]
for _d in LAYER_DIMS:
    LAYER_INDICES.append(LAYER_INDICES[-1] + _d)
TOTAL_ROWS = sum(LAYER_DIMS[1:])

NC = 2
NS = 16
LANES = 16
PAD = 72

_LAYERS = []
for _i in range(1, len(LAYER_DIMS)):
    _LAYERS.append((LAYER_DIMS[_i], LAYER_INDICES[_i - 1],
                    LAYER_INDICES[_i] - INPUT_DIM,
                    LAYER_INDICES[_i] - INPUT_DIM))

_MAX_NPT = max(n for n, _, _, _ in _LAYERS) // NS


def _build_net_kernel():
    mesh = plsc.VectorSubcoreMesh(core_axis_name="c", subcore_axis_name="s")
    out_type = [
        jax.ShapeDtypeStruct((NC, HIDDEN_DIMS[0]), jnp.float32),
        jax.ShapeDtypeStruct((NC, HIDDEN_DIMS[1]), jnp.float32),
        jax.ShapeDtypeStruct((OUTPUT_DIM,), jnp.float32),
    ]

    @functools.partial(
        pl.kernel,
        out_type=out_type,
        mesh=mesh,
        compiler_params=pltpu.CompilerParams(needs_layout_passes=False,
                                             use_tc_tiling_on_sc=False),
        scratch_types=[
            pltpu.VMEM((_MAX_NPT, PAD), jnp.int32),
            pltpu.VMEM((_MAX_NPT, PAD), jnp.float32),
            pltpu.VMEM((HIDDEN_DIMS[0],), jnp.float32),
            pltpu.VMEM((TOTAL_ROWS,), jnp.float32),
            pltpu.VMEM((_MAX_NPT,), jnp.float32),
            pltpu.SemaphoreType.DMA,
            pltpu.SemaphoreType.DMA,
            pltpu.SemaphoreType.DMA,
            pltpu.SemaphoreType.DMA,
        ],
    )
    def net(x_hbm, src_hbm, w_hbm, b_hbm, st1, st2, out_hbm,
            src_v, w_v, acts_v, bias_v, out_v, s0, s1, s2, s3):
        c = lax.axis_index("c")
        s = lax.axis_index("s")
        lane = lax.iota(jnp.int32, LANES)
        zero = jnp.zeros((LANES,), jnp.float32)

        def load_edges(row0, npt):
            c1 = pltpu.async_copy(
                src_hbm.at[pl.ds(row0, npt), :],
                src_v.at[pl.ds(0, npt), pl.ds(0, FANIN)], s0)
            c2 = pltpu.async_copy(
                w_hbm.at[pl.ds(row0, npt), :],
                w_v.at[pl.ds(0, npt), pl.ds(0, FANIN)], s1)
            return c1, c2

        def compute(npt, prev_start, bias_base):
            def group(g, _):
                rows = g * LANES + lane

                def jblock(t, accs):
                    a0, a1, a2, a3 = accs
                    accs = [a0, a1, a2, a3]
                    for u in range(8):
                        cols = jnp.full((LANES,), 0, jnp.int32) + (t * 8 + u)
                        si = plsc.load_gather(src_v, [rows, cols])
                        wv = plsc.load_gather(w_v, [rows, cols])
                        av = plsc.load_gather(acts_v, [si - prev_start])
                        accs[u % 4] = accs[u % 4] + av * wv
                    return tuple(accs)

                accs = lax.fori_loop(0, FANIN // 8, jblock,
                                     (zero, zero, zero, zero))
                acc = (accs[0] + accs[1]) + (accs[2] + accs[3])
                b = bias_v[pl.ds(bias_base + g * LANES, LANES)]
                out_v[pl.ds(g * LANES, LANES)] = jnp.maximum(acc + b, 0.0)
                return 0

            lax.fori_loop(0, npt // LANES, group, 0)

        n1, ps1, er1, bb1 = _LAYERS[0]
        npt1 = n1 // NS
        cx = pltpu.async_copy(x_hbm, acts_v.at[pl.ds(0, INPUT_DIM)], s2)
        cb = pltpu.async_copy(b_hbm, bias_v, s3)
        e1a, e1b = load_edges(er1 + s * npt1, npt1)
        cx.wait()
        cb.wait()
        e1a.wait()
        e1b.wait()
        compute(npt1, ps1, bb1 + s * npt1)
        co = pltpu.async_copy(out_v, st1.at[c, pl.ds(s * npt1, npt1)], s2)
        co.wait()
        plsc.subcore_barrier()

        n2, ps2, er2, bb2 = _LAYERS[1]
        npt2 = n2 // NS
        e2a, e2b = load_edges(er2 + s * npt2, npt2)
        ca = pltpu.async_copy(st1.at[c], acts_v, s2)
        ca.wait()
        e2a.wait()
        e2b.wait()
        compute(npt2, ps2, bb2 + s * npt2)
        co = pltpu.async_copy(out_v, st2.at[c, pl.ds(s * npt2, npt2)], s2)
        co.wait()
        plsc.subcore_barrier()

        n3, ps3, er3, bb3 = _LAYERS[2]
        npt3 = n3 // NS
        e3a, e3b = load_edges(er3 + s * npt3, npt3)
        ca = pltpu.async_copy(st2.at[c], acts_v, s2)
        ca.wait()
        e3a.wait()
        e3b.wait()
        compute(npt3, ps3, bb3 + s * npt3)

        @pl.when(c == 0)
        def _():
            pltpu.async_copy(out_v.at[pl.ds(0, npt3)],
                             out_hbm.at[pl.ds(s * npt3, npt3)], s2).wait()

    return net


_NET = _build_net_kernel()


def kernel(x, edge_index, weights, bias):
    src2d = edge_index[0].reshape(TOTAL_ROWS, FANIN)
    w2d = weights.reshape(TOTAL_ROWS, FANIN)
    _, _, out = _NET(x, src2d, w2d, bias)
    return out
